# Initial kernel scaffold; baseline (speedup 1.0000x reference)
import jax
import jax.numpy as jnp
from jax.experimental import pallas as pl


def kernel(x, pos, edge_index, edge_attr, emb_w, emb_b, edge_w1, edge_b1, edge_w2, edge_b2, att_w, att_b, node_w1, node_b1, node_w2, node_b2, ln_g, ln_b, out_w, out_b):
    raise NotImplementedError("write your pallas kernel here")


# speedup vs baseline: 2.2461x; 2.2461x over previous
import functools

import jax
import jax.numpy as jnp
from jax import lax
from jax.experimental import pallas as pl
from jax.experimental.pallas import tpu as pltpu
from jax.experimental.pallas import tpu_sc as plsc

N = 10000
E = 320000
D = 128
DE = 16
L = 4
TW = 144
NB = 1000
EB = 2000
CH = 128
NCHUNK = E // CH
NW = 32
NPAD = 10240

_silu = jax.nn.silu


def _init_body(x_ref, pos_ref, ew_ref, eb_ref, w1a_ref, w1b_ref, b1_ref,
               h_ref, p_ref, r_ref):
    h = jnp.maximum(x_ref[...] @ ew_ref[...] + eb_ref[...], 0.0)
    h_ref[...] = h
    pp = jnp.concatenate(
        [pos_ref[...], jnp.zeros((NB, TW - D - 3), jnp.float32)], axis=1)
    p_ref[...] = jnp.concatenate([h @ w1a_ref[...] + b1_ref[...], pp], axis=1)
    r_ref[...] = jnp.concatenate([h @ w1b_ref[...], -pp], axis=1)


_full = lambda shape: pl.BlockSpec(shape, lambda i: (0,) * len(shape))
_rows = lambda shape: pl.BlockSpec(shape, lambda i: (i,) + (0,) * (len(shape) - 1))

_init_call = pl.pallas_call(
    _init_body,
    grid=(N // NB,),
    in_specs=[_rows((NB, D)), _rows((NB, 3)), _full((D, D)), _full((1, D)),
              _full((D, D)), _full((D, D)), _full((1, D))],
    out_specs=[_rows((NB, D)), _rows((NB, TW)), _rows((NB, TW))],
    out_shape=[jax.ShapeDtypeStruct((N, D), jnp.float32),
               jax.ShapeDtypeStruct((N, TW), jnp.float32),
               jax.ShapeDtypeStruct((N, TW), jnp.float32)],
)


_sc_mesh = plsc.VectorSubcoreMesh(core_axis_name="c", subcore_axis_name="s")


@functools.partial(
    pl.kernel,
    mesh=_sc_mesh,
    out_type=jax.ShapeDtypeStruct((E, TW), jnp.float32),
    compiler_params=pltpu.CompilerParams(use_tc_tiling_on_sc=False),
    scratch_types=[
        pltpu.VMEM((CH,), jnp.int32),
        pltpu.VMEM((CH,), jnp.int32),
        pltpu.VMEM((CH, TW), jnp.float32),
        pltpu.VMEM((CH, TW), jnp.float32),
        pltpu.SemaphoreType.DMA,
        pltpu.SemaphoreType.DMA,
    ],
)
def _gather_call(p_hbm, r_hbm, dst_hbm, src_hbm, g_hbm,
                 di_v, si_v, pb_v, rb_v, sem0, sem1):
    wid = lax.axis_index("s") * 2 + lax.axis_index("c")

    def body(i, carry):
        c = wid + i * NW

        @pl.when(c < NCHUNK)
        def _():
            off = c * CH
            pltpu.sync_copy(dst_hbm.at[pl.ds(off, CH)], di_v)
            pltpu.sync_copy(src_hbm.at[pl.ds(off, CH)], si_v)
            cp = pltpu.make_async_copy(p_hbm.at[di_v], pb_v, sem0)
            cr = pltpu.make_async_copy(r_hbm.at[si_v], rb_v, sem1)
            cp.start()
            cr.start()
            cp.wait()
            cr.wait()

            def add_row(rr, carry):
                for s in range(TW // 16):
                    sl = pl.ds(s * 16, 16)
                    pb_v[rr, sl] = pb_v[rr, sl] + rb_v[rr, sl]
                return carry

            lax.fori_loop(0, CH, add_row, 0)
            pltpu.sync_copy(pb_v, g_hbm.at[pl.ds(off, CH)])

        return carry

    lax.fori_loop(0, (NCHUNK + NW - 1) // NW, body, 0)


@functools.partial(
    pl.kernel,
    mesh=_sc_mesh,
    out_type=jax.ShapeDtypeStruct((2, NPAD, D), jnp.float32),
    scratch_types=[
        pltpu.VMEM((CH,), jnp.int32),
        pltpu.VMEM((CH, D), jnp.float32),
        pltpu.VMEM_SHARED((NPAD, D), jnp.float32),
    ],
)
def _scatter_call(ma_hbm, dst_hbm, zero_hbm, out_hbm, di_v, rb_v, acc_sh):
    cid = lax.axis_index("c")
    sid = lax.axis_index("s")
    rows = NPAD // 16
    sl_mine = pl.ds(sid * rows, rows)
    pltpu.sync_copy(zero_hbm.at[sl_mine], acc_sh.at[sl_mine])
    plsc.subcore_barrier()

    half = NCHUNK // 2

    def body(i, carry):
        j = sid + i * 16

        @pl.when(j < half)
        def _():
            off = (cid * half + j) * CH
            pltpu.sync_copy(dst_hbm.at[pl.ds(off, CH)], di_v)
            pltpu.sync_copy(ma_hbm.at[pl.ds(off, CH)], rb_v)
            pltpu.sync_copy(rb_v, acc_sh.at[di_v], add=True)

        return carry

    lax.fori_loop(0, (half + 15) // 16, body, 0)
    plsc.subcore_barrier()
    pltpu.sync_copy(acc_sh.at[sl_mine], out_hbm.at[cid, sl_mine])


def _edge_body(g_ref, ea_ref, w1c_ref, w1d_ref, w2_ref, b2_ref,
               aw_ref, ab_ref, ma_ref):
    g = g_ref[...]
    rel = g[:, D:D + 3]
    dist = jnp.sqrt(jnp.sum(rel * rel, axis=1, keepdims=True) + 1e-12)
    pre = g[:, :D] + dist * w1c_ref[...] + ea_ref[...] @ w1d_ref[...]
    m = _silu(_silu(pre) @ w2_ref[...] + b2_ref[...])
    a = jax.nn.sigmoid(m @ aw_ref[...] + ab_ref[...])
    ma_ref[...] = m * a


_edge_call = pl.pallas_call(
    _edge_body,
    grid=(E // EB,),
    in_specs=[_rows((EB, TW)), _rows((EB, DE)), _full((1, D)), _full((DE, D)),
              _full((D, D)), _full((1, D)), _full((D, 1)), _full((1, 1))],
    out_specs=_rows((EB, D)),
    out_shape=jax.ShapeDtypeStruct((E, D), jnp.float32),
)


def _node_common(h_ref, p0_ref, p1_ref, nw1h_ref, nw1a_ref, nb1_ref,
                 nw2_ref, nb2_ref, g_ref, b_ref):
    h = h_ref[...]
    aggr = p0_ref[...] + p1_ref[...]
    nin1 = h @ nw1h_ref[...] + aggr @ nw1a_ref[...] + nb1_ref[...]
    u = h + _silu(nin1) @ nw2_ref[...] + nb2_ref[...]
    mu = jnp.mean(u, axis=1, keepdims=True)
    var = jnp.mean((u - mu) ** 2, axis=1, keepdims=True)
    return (u - mu) / jnp.sqrt(var + 1e-5) * g_ref[...] + b_ref[...]


def _node_mid_body(h_ref, p0_ref, p1_ref, pos_ref, nw1h_ref, nw1a_ref,
                   nb1_ref, nw2_ref, nb2_ref, g_ref, b_ref,
                   w1a_ref, w1b_ref, b1_ref, h_out, p_out, r_out):
    hn = _node_common(h_ref, p0_ref, p1_ref, nw1h_ref, nw1a_ref, nb1_ref,
                      nw2_ref, nb2_ref, g_ref, b_ref)
    h_out[...] = hn
    pp = jnp.concatenate(
        [pos_ref[...], jnp.zeros((NB, TW - D - 3), jnp.float32)], axis=1)
    p_out[...] = jnp.concatenate([hn @ w1a_ref[...] + b1_ref[...], pp], axis=1)
    r_out[...] = jnp.concatenate([hn @ w1b_ref[...], -pp], axis=1)


_node_mid_call = pl.pallas_call(
    _node_mid_body,
    grid=(N // NB,),
    in_specs=[_rows((NB, D)), _rows((NB, D)), _rows((NB, D)), _rows((NB, 3)),
              _full((D, D)), _full((D, D)), _full((1, D)), _full((D, D)),
              _full((1, D)), _full((1, D)), _full((1, D)),
              _full((D, D)), _full((D, D)), _full((1, D))],
    out_specs=[_rows((NB, D)), _rows((NB, TW)), _rows((NB, TW))],
    out_shape=[jax.ShapeDtypeStruct((N, D), jnp.float32),
               jax.ShapeDtypeStruct((N, TW), jnp.float32),
               jax.ShapeDtypeStruct((N, TW), jnp.float32)],
)


def _node_last_body(h_ref, p0_ref, p1_ref, nw1h_ref, nw1a_ref, nb1_ref,
                    nw2_ref, nb2_ref, g_ref, b_ref, ow_ref, ob_ref, y_out):
    hn = _node_common(h_ref, p0_ref, p1_ref, nw1h_ref, nw1a_ref, nb1_ref,
                      nw2_ref, nb2_ref, g_ref, b_ref)
    y_out[...] = hn @ ow_ref[...] + ob_ref[...]


_node_last_call = pl.pallas_call(
    _node_last_body,
    grid=(N // NB,),
    in_specs=[_rows((NB, D)), _rows((NB, D)), _rows((NB, D)),
              _full((D, D)), _full((D, D)), _full((1, D)), _full((D, D)),
              _full((1, D)), _full((1, D)), _full((1, D)),
              _full((D, 1)), _full((1, 1))],
    out_specs=_rows((NB, 1)),
    out_shape=jax.ShapeDtypeStruct((N, 1), jnp.float32),
)


def kernel(x, pos, edge_index, edge_attr, emb_w, emb_b, edge_w1, edge_b1,
           edge_w2, edge_b2, att_w, att_b, node_w1, node_b1, node_w2,
           node_b2, ln_g, ln_b, out_w, out_b):
    src = edge_index[0]
    dst = edge_index[1]
    zeros = jnp.zeros((NPAD, D), jnp.float32)

    h, P, R = _init_call(x, pos, emb_w, emb_b.reshape(1, D),
                         edge_w1[0, :D], edge_w1[0, D:2 * D],
                         edge_b1[0].reshape(1, D))
    for l in range(L):
        G = _gather_call(P, R, dst, src)
        ma = _edge_call(G, edge_attr,
                        edge_w1[l, 2 * D].reshape(1, D),
                        edge_w1[l, 2 * D + 1:],
                        edge_w2[l], edge_b2[l].reshape(1, D),
                        att_w[l], att_b[l].reshape(1, 1))
        parts = _scatter_call(ma, dst, zeros)
        if l < L - 1:
            h, P, R = _node_mid_call(
                h, parts[0], parts[1], pos,
                node_w1[l, :D], node_w1[l, D:], node_b1[l].reshape(1, D),
                node_w2[l], node_b2[l].reshape(1, D),
                ln_g[l].reshape(1, D), ln_b[l].reshape(1, D),
                edge_w1[l + 1, :D], edge_w1[l + 1, D:2 * D],
                edge_b1[l + 1].reshape(1, D))
        else:
            y = _node_last_call(
                h, parts[0], parts[1],
                node_w1[l, :D], node_w1[l, D:], node_b1[l].reshape(1, D),
                node_w2[l], node_b2[l].reshape(1, D),
                ln_g[l].reshape(1, D), ln_b[l].reshape(1, D),
                out_w, out_b.reshape(1, 1))
    return y


# speedup vs baseline: 3.2646x; 1.4534x over previous
import functools

import jax
import jax.numpy as jnp
from jax import lax
from jax.experimental import pallas as pl
from jax.experimental.pallas import tpu as pltpu
from jax.experimental.pallas import tpu_sc as plsc

N = 10000
E = 320000
D = 128
DE = 16
L = 4
NB = 1000
EB = 2000
NW = 32
NPAD = 10240
SUP = 200
EPW = E // NW
NSUP = EPW // SUP
CH = 128
NCHUNK = E // CH

_silu = jax.nn.silu


def _init_body(x_ref, ew_ref, eb_ref, w1a_ref, w1b_ref, b1_ref,
               h_ref, p_ref, r_ref):
    h = jnp.maximum(x_ref[...] @ ew_ref[...] + eb_ref[...], 0.0)
    h_ref[...] = h
    p_ref[...] = h @ w1a_ref[...] + b1_ref[...]
    r_ref[...] = h @ w1b_ref[...]


_full = lambda shape: pl.BlockSpec(shape, lambda i: (0,) * len(shape))
_rows = lambda shape: pl.BlockSpec(shape, lambda i: (i,) + (0,) * (len(shape) - 1))

_init_call = pl.pallas_call(
    _init_body,
    grid=(N // NB,),
    in_specs=[_rows((NB, D)), _full((D, D)), _full((1, D)),
              _full((D, D)), _full((D, D)), _full((1, D))],
    out_specs=[_rows((NB, D)), _rows((NB, D)), _rows((NB, D))],
    out_shape=[jax.ShapeDtypeStruct((N, D), jnp.float32)] * 3,
)


_sc_mesh = plsc.VectorSubcoreMesh(core_axis_name="c", subcore_axis_name="s")


@functools.partial(
    pl.kernel,
    mesh=_sc_mesh,
    out_type=jax.ShapeDtypeStruct((E, D), jnp.float32),
    scratch_types=[
        pltpu.VMEM((128,), jnp.int32),
        pltpu.VMEM((72,), jnp.int32),
        pltpu.VMEM((128,), jnp.int32),
        pltpu.VMEM((72,), jnp.int32),
        pltpu.VMEM((SUP, D), jnp.float32),
        pltpu.VMEM((SUP, D), jnp.float32),
        pltpu.SemaphoreType.DMA,
    ],
)
def _relgather_call(posp_hbm, dst_hbm, src_hbm, rel_hbm,
                    dia, dib, sia, sib, ba, bb, sem):
    wid = lax.axis_index("s") * 2 + lax.axis_index("c")
    base = wid * EPW

    def body(j, carry):
        off = base + j * SUP
        pltpu.sync_copy(dst_hbm.at[pl.ds(off, 128)], dia)
        pltpu.sync_copy(dst_hbm.at[pl.ds(off + 128, 72)], dib)
        pltpu.sync_copy(src_hbm.at[pl.ds(off, 128)], sia)
        pltpu.sync_copy(src_hbm.at[pl.ds(off + 128, 72)], sib)
        cps = (pltpu.make_async_copy(posp_hbm.at[dia], ba.at[pl.ds(0, 128)], sem),
               pltpu.make_async_copy(posp_hbm.at[dib], ba.at[pl.ds(128, 72)], sem),
               pltpu.make_async_copy(posp_hbm.at[sia], bb.at[pl.ds(0, 128)], sem),
               pltpu.make_async_copy(posp_hbm.at[sib], bb.at[pl.ds(128, 72)], sem))
        for cp in cps:
            cp.start()
        for cp in cps:
            cp.wait()

        def sub_row(r, c2):
            sl = pl.ds(0, 16)
            ba[r, sl] = ba[r, sl] - bb[r, sl]
            return c2

        lax.fori_loop(0, SUP, sub_row, 0)
        pltpu.sync_copy(ba, rel_hbm.at[pl.ds(off, SUP)])
        return carry

    lax.fori_loop(0, NSUP, body, 0)


@functools.partial(
    pl.kernel,
    mesh=_sc_mesh,
    out_type=jax.ShapeDtypeStruct((E, D), jnp.float32),
    scratch_types=(
        [pltpu.VMEM((128,), jnp.int32), pltpu.VMEM((72,), jnp.int32),
         pltpu.VMEM((128,), jnp.int32), pltpu.VMEM((72,), jnp.int32)] * 2
        + [pltpu.VMEM((SUP, D), jnp.float32)] * 4
        + [pltpu.SemaphoreType.DMA] * 6
    ),
)
def _gather_call(p_hbm, r_hbm, dst_hbm, src_hbm, g_hbm,
                 dia0, dib0, sia0, sib0, dia1, dib1, sia1, sib1,
                 pb0, pb1, rb0, rb1, is0, is1, gs0, gs1, ws0, ws1):
    wid = lax.axis_index("s") * 2 + lax.axis_index("c")
    base = wid * EPW
    idx = ((dia0, dib0, sia0, sib0), (dia1, dib1, sia1, sib1))
    pbs = (pb0, pb1)
    rbs = (rb0, rb1)
    isems = (is0, is1)
    gsems = (gs0, gs1)
    wsems = (ws0, ws1)

    def idx_copies(b, j):
        off = base + j * SUP
        a = idx[b]
        return (pltpu.make_async_copy(dst_hbm.at[pl.ds(off, 128)], a[0], isems[b]),
                pltpu.make_async_copy(dst_hbm.at[pl.ds(off + 128, 72)], a[1], isems[b]),
                pltpu.make_async_copy(src_hbm.at[pl.ds(off, 128)], a[2], isems[b]),
                pltpu.make_async_copy(src_hbm.at[pl.ds(off + 128, 72)], a[3], isems[b]))

    def load_idx(b, j):
        for cp in idx_copies(b, j):
            cp.start()

    def wait_idx(b):
        for cp in idx_copies(b, 0):
            cp.wait()

    def gather_copies(b):
        a = idx[b]
        return (pltpu.make_async_copy(p_hbm.at[a[0]], pbs[b].at[pl.ds(0, 128)], gsems[b]),
                pltpu.make_async_copy(p_hbm.at[a[1]], pbs[b].at[pl.ds(128, 72)], gsems[b]),
                pltpu.make_async_copy(r_hbm.at[a[2]], rbs[b].at[pl.ds(0, 128)], gsems[b]),
                pltpu.make_async_copy(r_hbm.at[a[3]], rbs[b].at[pl.ds(128, 72)], gsems[b]))

    def wb(b, j):
        return pltpu.make_async_copy(
            pbs[b], g_hbm.at[pl.ds(base + j * SUP, SUP)], wsems[b])

    load_idx(0, 0)
    wait_idx(0)
    for cp in gather_copies(0):
        cp.start()
    load_idx(1, 1)

    def pair(t, carry):
        for b in (0, 1):
            j = 2 * t + b
            nb = 1 - b
            for cp in gather_copies(b):
                cp.wait()

            def add_row(r, c2):
                for s in range(D // 16):
                    sl = pl.ds(s * 16, 16)
                    pbs[b][r, sl] = pbs[b][r, sl] + rbs[b][r, sl]
                return c2

            lax.fori_loop(0, SUP, add_row, 0)
            wb(b, j).start()

            @pl.when(j + 1 < NSUP)
            def _():
                wait_idx(nb)

                @pl.when(j >= 1)
                def _():
                    wb(nb, 0).wait()

                for cp in gather_copies(nb):
                    cp.start()

            @pl.when(j + 2 < NSUP)
            def _():
                load_idx(b, j + 2)

        return carry

    lax.fori_loop(0, NSUP // 2, pair, 0)
    wb(0, 0).wait()
    wb(1, 0).wait()


@functools.partial(
    pl.kernel,
    mesh=_sc_mesh,
    out_type=jax.ShapeDtypeStruct((2, NPAD, D), jnp.float32),
    scratch_types=[
        pltpu.VMEM((CH,), jnp.int32),
        pltpu.VMEM((CH, D), jnp.float32),
        pltpu.VMEM_SHARED((NPAD, D), jnp.float32),
    ],
)
def _scatter_call(ma_hbm, dst_hbm, zero_hbm, out_hbm, di_v, rb_v, acc_sh):
    cid = lax.axis_index("c")
    sid = lax.axis_index("s")
    rows = NPAD // 16
    sl_mine = pl.ds(sid * rows, rows)
    pltpu.sync_copy(zero_hbm.at[sl_mine], acc_sh.at[sl_mine])
    plsc.subcore_barrier()

    half = NCHUNK // 2

    def body(i, carry):
        j = sid + i * 16

        @pl.when(j < half)
        def _():
            off = (cid * half + j) * CH
            pltpu.sync_copy(dst_hbm.at[pl.ds(off, CH)], di_v)
            pltpu.sync_copy(ma_hbm.at[pl.ds(off, CH)], rb_v)
            pltpu.sync_copy(rb_v, acc_sh.at[di_v], add=True)

        return carry

    lax.fori_loop(0, (half + 15) // 16, body, 0)
    plsc.subcore_barrier()
    pltpu.sync_copy(acc_sh.at[sl_mine], out_hbm.at[cid, sl_mine])


def _edge_core(pre0, ea, w1c_ref, w1d_ref, w2_ref, b2_ref, aw_ref, ab_ref,
               dist):
    pre = pre0 + dist * w1c_ref[...] + ea @ w1d_ref[...]
    m = _silu(_silu(pre) @ w2_ref[...] + b2_ref[...])
    a = jax.nn.sigmoid(m @ aw_ref[...] + ab_ref[...])
    return m * a


def _edge0_body(g_ref, rel_ref, ea_ref, w1c_ref, w1d_ref, w2_ref, b2_ref,
                aw_ref, ab_ref, ma_ref, dist_ref):
    rel = rel_ref[...]
    dist = jnp.sqrt(jnp.sum(rel * rel, axis=1, keepdims=True) + 1e-12)
    dist_ref[...] = dist
    ma_ref[...] = _edge_core(g_ref[...], ea_ref[...], w1c_ref, w1d_ref,
                             w2_ref, b2_ref, aw_ref, ab_ref, dist)


_edge0_call = pl.pallas_call(
    _edge0_body,
    grid=(E // EB,),
    in_specs=[_rows((EB, D)), _rows((EB, D)), _rows((EB, DE)), _full((1, D)),
              _full((DE, D)), _full((D, D)), _full((1, D)), _full((D, 1)),
              _full((1, 1))],
    out_specs=[_rows((EB, D)), _rows((EB, 1))],
    out_shape=[jax.ShapeDtypeStruct((E, D), jnp.float32),
               jax.ShapeDtypeStruct((E, 1), jnp.float32)],
)


def _edge_body(g_ref, dist_ref, ea_ref, w1c_ref, w1d_ref, w2_ref, b2_ref,
               aw_ref, ab_ref, ma_ref):
    ma_ref[...] = _edge_core(g_ref[...], ea_ref[...], w1c_ref, w1d_ref,
                             w2_ref, b2_ref, aw_ref, ab_ref, dist_ref[...])


_edge_call = pl.pallas_call(
    _edge_body,
    grid=(E // EB,),
    in_specs=[_rows((EB, D)), _rows((EB, 1)), _rows((EB, DE)), _full((1, D)),
              _full((DE, D)), _full((D, D)), _full((1, D)), _full((D, 1)),
              _full((1, 1))],
    out_specs=_rows((EB, D)),
    out_shape=jax.ShapeDtypeStruct((E, D), jnp.float32),
)


def _node_common(h_ref, p0_ref, p1_ref, nw1h_ref, nw1a_ref, nb1_ref,
                 nw2_ref, nb2_ref, g_ref, b_ref):
    h = h_ref[...]
    aggr = p0_ref[...] + p1_ref[...]
    nin1 = h @ nw1h_ref[...] + aggr @ nw1a_ref[...] + nb1_ref[...]
    u = h + _silu(nin1) @ nw2_ref[...] + nb2_ref[...]
    mu = jnp.mean(u, axis=1, keepdims=True)
    var = jnp.mean((u - mu) ** 2, axis=1, keepdims=True)
    return (u - mu) / jnp.sqrt(var + 1e-5) * g_ref[...] + b_ref[...]


def _node_mid_body(h_ref, p0_ref, p1_ref, nw1h_ref, nw1a_ref,
                   nb1_ref, nw2_ref, nb2_ref, g_ref, b_ref,
                   w1a_ref, w1b_ref, b1_ref, h_out, p_out, r_out):
    hn = _node_common(h_ref, p0_ref, p1_ref, nw1h_ref, nw1a_ref, nb1_ref,
                      nw2_ref, nb2_ref, g_ref, b_ref)
    h_out[...] = hn
    p_out[...] = hn @ w1a_ref[...] + b1_ref[...]
    r_out[...] = hn @ w1b_ref[...]


_node_mid_call = pl.pallas_call(
    _node_mid_body,
    grid=(N // NB,),
    in_specs=[_rows((NB, D)), _rows((NB, D)), _rows((NB, D)),
              _full((D, D)), _full((D, D)), _full((1, D)), _full((D, D)),
              _full((1, D)), _full((1, D)), _full((1, D)),
              _full((D, D)), _full((D, D)), _full((1, D))],
    out_specs=[_rows((NB, D)), _rows((NB, D)), _rows((NB, D))],
    out_shape=[jax.ShapeDtypeStruct((N, D), jnp.float32)] * 3,
)


def _node_last_body(h_ref, p0_ref, p1_ref, nw1h_ref, nw1a_ref, nb1_ref,
                    nw2_ref, nb2_ref, g_ref, b_ref, ow_ref, ob_ref, y_out):
    hn = _node_common(h_ref, p0_ref, p1_ref, nw1h_ref, nw1a_ref, nb1_ref,
                      nw2_ref, nb2_ref, g_ref, b_ref)
    y_out[...] = hn @ ow_ref[...] + ob_ref[...]


_node_last_call = pl.pallas_call(
    _node_last_body,
    grid=(N // NB,),
    in_specs=[_rows((NB, D)), _rows((NB, D)), _rows((NB, D)),
              _full((D, D)), _full((D, D)), _full((1, D)), _full((D, D)),
              _full((1, D)), _full((1, D)), _full((1, D)),
              _full((D, 1)), _full((1, 1))],
    out_specs=_rows((NB, 1)),
    out_shape=jax.ShapeDtypeStruct((N, 1), jnp.float32),
)


def kernel(x, pos, edge_index, edge_attr, emb_w, emb_b, edge_w1, edge_b1,
           edge_w2, edge_b2, att_w, att_b, node_w1, node_b1, node_w2,
           node_b2, ln_g, ln_b, out_w, out_b):
    src = edge_index[0]
    dst = edge_index[1]
    zeros = jnp.zeros((NPAD, D), jnp.float32)
    posp = jnp.pad(pos, ((0, 0), (0, D - 3)))

    relw = _relgather_call(posp, dst, src)
    h, P, R = _init_call(x, emb_w, emb_b.reshape(1, D),
                         edge_w1[0, :D], edge_w1[0, D:2 * D],
                         edge_b1[0].reshape(1, D))
    dist = None
    for l in range(L):
        G = _gather_call(P, R, dst, src)
        eargs = (edge_w1[l, 2 * D].reshape(1, D), edge_w1[l, 2 * D + 1:],
                 edge_w2[l], edge_b2[l].reshape(1, D),
                 att_w[l], att_b[l].reshape(1, 1))
        if l == 0:
            ma, dist = _edge0_call(G, relw, edge_attr, *eargs)
        else:
            ma = _edge_call(G, dist, edge_attr, *eargs)
        parts = _scatter_call(ma, dst, zeros)
        if l < L - 1:
            h, P, R = _node_mid_call(
                h, parts[0], parts[1],
                node_w1[l, :D], node_w1[l, D:], node_b1[l].reshape(1, D),
                node_w2[l], node_b2[l].reshape(1, D),
                ln_g[l].reshape(1, D), ln_b[l].reshape(1, D),
                edge_w1[l + 1, :D], edge_w1[l + 1, D:2 * D],
                edge_b1[l + 1].reshape(1, D))
        else:
            y = _node_last_call(
                h, parts[0], parts[1],
                node_w1[l, :D], node_w1[l, D:], node_b1[l].reshape(1, D),
                node_w2[l], node_b2[l].reshape(1, D),
                ln_g[l].reshape(1, D), ln_b[l].reshape(1, D),
                out_w, out_b.reshape(1, 1))
    return y


# speedup vs baseline: 3.8795x; 1.1884x over previous
import functools

import jax
import jax.numpy as jnp
from jax import lax
from jax.experimental import pallas as pl
from jax.experimental.pallas import tpu as pltpu
from jax.experimental.pallas import tpu_sc as plsc

N = 10000
E = 320000
E2 = E // 2
D = 128
DE = 16
L = 4
NB = 1000
EB = 2000
NW = 32
NPAD = 10240
SUP = 200
CH = 128

_silu = jax.nn.silu


def _init_body(x_ref, ew_ref, eb_ref, w1a_ref, w1b_ref, b1_ref,
               h_ref, p_ref, r_ref):
    h = jnp.maximum(x_ref[...] @ ew_ref[...] + eb_ref[...], 0.0)
    h_ref[...] = h
    p_ref[...] = h @ w1a_ref[...] + b1_ref[...]
    r_ref[...] = h @ w1b_ref[...]


_full = lambda shape: pl.BlockSpec(shape, lambda i: (0,) * len(shape))
_rows = lambda shape: pl.BlockSpec(shape, lambda i: (i,) + (0,) * (len(shape) - 1))

_init_call = pl.pallas_call(
    _init_body,
    grid=(N // NB,),
    in_specs=[_rows((NB, D)), _full((D, D)), _full((1, D)),
              _full((D, D)), _full((D, D)), _full((1, D))],
    out_specs=[_rows((NB, D)), _rows((NB, D)), _rows((NB, D))],
    out_shape=[jax.ShapeDtypeStruct((N, D), jnp.float32)] * 3,
)


_sc_mesh = plsc.VectorSubcoreMesh(core_axis_name="c", subcore_axis_name="s")


@functools.partial(
    pl.kernel,
    mesh=_sc_mesh,
    out_type=jax.ShapeDtypeStruct((E, D), jnp.float32),
    scratch_types=[
        pltpu.VMEM((128,), jnp.int32),
        pltpu.VMEM((72,), jnp.int32),
        pltpu.VMEM((128,), jnp.int32),
        pltpu.VMEM((72,), jnp.int32),
        pltpu.VMEM((SUP, D), jnp.float32),
        pltpu.VMEM((SUP, D), jnp.float32),
        pltpu.SemaphoreType.DMA,
    ],
)
def _relgather_call(posp_hbm, dst_hbm, src_hbm, rel_hbm,
                    dia, dib, sia, sib, ba, bb, sem):
    wid = lax.axis_index("s") * 2 + lax.axis_index("c")
    base = wid * (E // NW)

    def body(j, carry):
        off = base + j * SUP
        pltpu.sync_copy(dst_hbm.at[pl.ds(off, 128)], dia)
        pltpu.sync_copy(dst_hbm.at[pl.ds(off + 128, 72)], dib)
        pltpu.sync_copy(src_hbm.at[pl.ds(off, 128)], sia)
        pltpu.sync_copy(src_hbm.at[pl.ds(off + 128, 72)], sib)
        cps = (pltpu.make_async_copy(posp_hbm.at[dia], ba.at[pl.ds(0, 128)], sem),
               pltpu.make_async_copy(posp_hbm.at[dib], ba.at[pl.ds(128, 72)], sem),
               pltpu.make_async_copy(posp_hbm.at[sia], bb.at[pl.ds(0, 128)], sem),
               pltpu.make_async_copy(posp_hbm.at[sib], bb.at[pl.ds(128, 72)], sem))
        for cp in cps:
            cp.start()
        for cp in cps:
            cp.wait()

        def sub_row(r, c2):
            sl = pl.ds(0, 16)
            ba[r, sl] = ba[r, sl] - bb[r, sl]
            return c2

        lax.fori_loop(0, SUP, sub_row, 0)
        pltpu.sync_copy(ba, rel_hbm.at[pl.ds(off, SUP)])
        return carry

    lax.fori_loop(0, E // NW // SUP, body, 0)


def _make_gather(b0):
    epw = E2 // NW
    ns = epw // SUP

    @functools.partial(
        pl.kernel,
        mesh=_sc_mesh,
        out_type=jax.ShapeDtypeStruct((E2, D), jnp.float32),
        scratch_types=(
            [pltpu.VMEM((128,), jnp.int32), pltpu.VMEM((72,), jnp.int32),
             pltpu.VMEM((128,), jnp.int32), pltpu.VMEM((72,), jnp.int32)] * 2
            + [pltpu.VMEM((SUP, D), jnp.float32)] * 4
            + [pltpu.SemaphoreType.DMA] * 6
        ),
    )
    def gather(p_hbm, r_hbm, dst_hbm, src_hbm, g_hbm,
               dia0, dib0, sia0, sib0, dia1, dib1, sia1, sib1,
               pb0, pb1, rb0, rb1, is0, is1, gs0, gs1, ws0, ws1):
        wid = lax.axis_index("s") * 2 + lax.axis_index("c")
        wbase = wid * epw
        idx = ((dia0, dib0, sia0, sib0), (dia1, dib1, sia1, sib1))
        pbs = (pb0, pb1)
        rbs = (rb0, rb1)
        isems = (is0, is1)
        gsems = (gs0, gs1)
        wsems = (ws0, ws1)

        def idx_copies(b, j):
            off = b0 + wbase + j * SUP
            a = idx[b]
            return (pltpu.make_async_copy(dst_hbm.at[pl.ds(off, 128)], a[0], isems[b]),
                    pltpu.make_async_copy(dst_hbm.at[pl.ds(off + 128, 72)], a[1], isems[b]),
                    pltpu.make_async_copy(src_hbm.at[pl.ds(off, 128)], a[2], isems[b]),
                    pltpu.make_async_copy(src_hbm.at[pl.ds(off + 128, 72)], a[3], isems[b]))

        def load_idx(b, j):
            for cp in idx_copies(b, j):
                cp.start()

        def wait_idx(b):
            for cp in idx_copies(b, 0):
                cp.wait()

        def gather_copies(b):
            a = idx[b]
            return (pltpu.make_async_copy(p_hbm.at[a[0]], pbs[b].at[pl.ds(0, 128)], gsems[b]),
                    pltpu.make_async_copy(p_hbm.at[a[1]], pbs[b].at[pl.ds(128, 72)], gsems[b]),
                    pltpu.make_async_copy(r_hbm.at[a[2]], rbs[b].at[pl.ds(0, 128)], gsems[b]),
                    pltpu.make_async_copy(r_hbm.at[a[3]], rbs[b].at[pl.ds(128, 72)], gsems[b]))

        def wb(b, j):
            return pltpu.make_async_copy(
                pbs[b], g_hbm.at[pl.ds(wbase + j * SUP, SUP)], wsems[b])

        def add_rows(b):
            def add_row(r, c2):
                for q in range(2):
                    for s in range(D // 16):
                        sl = pl.ds(s * 16, 16)
                        pbs[b][2 * r + q, sl] = (pbs[b][2 * r + q, sl]
                                                 + rbs[b][2 * r + q, sl])
                return c2

            lax.fori_loop(0, SUP // 2, add_row, 0)

        load_idx(0, 0)
        wait_idx(0)
        for cp in gather_copies(0):
            cp.start()
        load_idx(1, 1)

        def pair(t, carry):
            for b in (0, 1):
                j = 2 * t + b
                nb = 1 - b
                for cp in gather_copies(b):
                    cp.wait()
                add_rows(b)
                wb(b, j).start()

                @pl.when(j + 1 < ns)
                def _():
                    wait_idx(nb)

                    @pl.when(j >= 1)
                    def _():
                        wb(nb, 0).wait()

                    for cp in gather_copies(nb):
                        cp.start()

                @pl.when(j + 2 < ns)
                def _():
                    load_idx(b, j + 2)

            return carry

        lax.fori_loop(0, ns // 2, pair, 0)
        if ns % 2 == 1:
            for cp in gather_copies(0):
                cp.wait()
            add_rows(0)
            wb(0, ns - 1).start()
        wb(0, 0).wait()
        wb(1, 0).wait()

    return gather


_gather_calls = [_make_gather(0), _make_gather(E2)]


def _make_scatter(b0):
    nchunk = E2 // CH
    half = nchunk // 2

    @functools.partial(
        pl.kernel,
        mesh=_sc_mesh,
        out_type=jax.ShapeDtypeStruct((2, NPAD, D), jnp.float32),
        scratch_types=[
            pltpu.VMEM((CH,), jnp.int32),
            pltpu.VMEM((CH, D), jnp.float32),
            pltpu.VMEM_SHARED((NPAD, D), jnp.float32),
        ],
    )
    def scatter(ma_hbm, dst_hbm, zero_hbm, out_hbm, di_v, rb_v, acc_sh):
        cid = lax.axis_index("c")
        sid = lax.axis_index("s")
        rows = NPAD // 16
        sl_mine = pl.ds(sid * rows, rows)
        pltpu.sync_copy(zero_hbm.at[sl_mine], acc_sh.at[sl_mine])
        plsc.subcore_barrier()

        def body(i, carry):
            j = sid + i * 16

            @pl.when(j < half)
            def _():
                off = (cid * half + j) * CH
                pltpu.sync_copy(dst_hbm.at[pl.ds(b0 + off, CH)], di_v)
                pltpu.sync_copy(ma_hbm.at[pl.ds(off, CH)], rb_v)
                pltpu.sync_copy(rb_v, acc_sh.at[di_v], add=True)

            return carry

        lax.fori_loop(0, (half + 15) // 16, body, 0)
        plsc.subcore_barrier()
        pltpu.sync_copy(acc_sh.at[sl_mine], out_hbm.at[cid, sl_mine])

    return scatter


_scatter_calls = [_make_scatter(0), _make_scatter(E2)]


def _edge_core(pre0, ea, w1c_ref, w1d_ref, w2_ref, b2_ref, aw_ref, ab_ref,
               dist):
    pre = pre0 + dist * w1c_ref[...] + ea @ w1d_ref[...]
    m = _silu(_silu(pre) @ w2_ref[...] + b2_ref[...])
    a = jax.nn.sigmoid(m @ aw_ref[...] + ab_ref[...])
    return m * a


def _make_edge_calls(h):
    off = h * (E2 // EB)
    _erows = lambda w: pl.BlockSpec((EB, w), lambda i: (off + i, 0))

    def edge0_body(g_ref, rel_ref, ea_ref, w1c_ref, w1d_ref, w2_ref, b2_ref,
                   aw_ref, ab_ref, ma_ref, dist_ref):
        rel = rel_ref[...]
        dist = jnp.sqrt(jnp.sum(rel * rel, axis=1, keepdims=True) + 1e-12)
        dist_ref[...] = dist
        ma_ref[...] = _edge_core(g_ref[...], ea_ref[...], w1c_ref, w1d_ref,
                                 w2_ref, b2_ref, aw_ref, ab_ref, dist)

    edge0 = pl.pallas_call(
        edge0_body,
        grid=(E2 // EB,),
        in_specs=[_rows((EB, D)), _erows(D), _erows(DE), _full((1, D)),
                  _full((DE, D)), _full((D, D)), _full((1, D)),
                  _full((D, 1)), _full((1, 1))],
        out_specs=[_rows((EB, D)), _rows((EB, 1))],
        out_shape=[jax.ShapeDtypeStruct((E2, D), jnp.float32),
                   jax.ShapeDtypeStruct((E2, 1), jnp.float32)],
    )

    def edge_body(g_ref, dist_ref, ea_ref, w1c_ref, w1d_ref, w2_ref, b2_ref,
                  aw_ref, ab_ref, ma_ref):
        ma_ref[...] = _edge_core(g_ref[...], ea_ref[...], w1c_ref, w1d_ref,
                                 w2_ref, b2_ref, aw_ref, ab_ref,
                                 dist_ref[...])

    edge = pl.pallas_call(
        edge_body,
        grid=(E2 // EB,),
        in_specs=[_rows((EB, D)), _rows((EB, 1)), _erows(DE), _full((1, D)),
                  _full((DE, D)), _full((D, D)), _full((1, D)),
                  _full((D, 1)), _full((1, 1))],
        out_specs=_rows((EB, D)),
        out_shape=jax.ShapeDtypeStruct((E2, D), jnp.float32),
    )

    return edge0, edge


_edge_calls = [_make_edge_calls(0), _make_edge_calls(1)]


def _node_common(h_ref, parts, nw1h_ref, nw1a_ref, nb1_ref,
                 nw2_ref, nb2_ref, g_ref, b_ref):
    h = h_ref[...]
    aggr = ((parts[0][...] + parts[1][...])
            + (parts[2][...] + parts[3][...]))
    nin1 = h @ nw1h_ref[...] + aggr @ nw1a_ref[...] + nb1_ref[...]
    u = h + _silu(nin1) @ nw2_ref[...] + nb2_ref[...]
    mu = jnp.mean(u, axis=1, keepdims=True)
    var = jnp.mean((u - mu) ** 2, axis=1, keepdims=True)
    return (u - mu) / jnp.sqrt(var + 1e-5) * g_ref[...] + b_ref[...]


def _node_mid_body(h_ref, p00, p01, p10, p11, nw1h_ref, nw1a_ref,
                   nb1_ref, nw2_ref, nb2_ref, g_ref, b_ref,
                   w1a_ref, w1b_ref, b1_ref, h_out, p_out, r_out):
    hn = _node_common(h_ref, (p00, p01, p10, p11), nw1h_ref, nw1a_ref,
                      nb1_ref, nw2_ref, nb2_ref, g_ref, b_ref)
    h_out[...] = hn
    p_out[...] = hn @ w1a_ref[...] + b1_ref[...]
    r_out[...] = hn @ w1b_ref[...]


_node_mid_call = pl.pallas_call(
    _node_mid_body,
    grid=(N // NB,),
    in_specs=[_rows((NB, D))] * 5 +
             [_full((D, D)), _full((D, D)), _full((1, D)), _full((D, D)),
              _full((1, D)), _full((1, D)), _full((1, D)),
              _full((D, D)), _full((D, D)), _full((1, D))],
    out_specs=[_rows((NB, D)), _rows((NB, D)), _rows((NB, D))],
    out_shape=[jax.ShapeDtypeStruct((N, D), jnp.float32)] * 3,
)


def _node_last_body(h_ref, p00, p01, p10, p11, nw1h_ref, nw1a_ref, nb1_ref,
                    nw2_ref, nb2_ref, g_ref, b_ref, ow_ref, ob_ref, y_out):
    hn = _node_common(h_ref, (p00, p01, p10, p11), nw1h_ref, nw1a_ref,
                      nb1_ref, nw2_ref, nb2_ref, g_ref, b_ref)
    y_out[...] = hn @ ow_ref[...] + ob_ref[...]


_node_last_call = pl.pallas_call(
    _node_last_body,
    grid=(N // NB,),
    in_specs=[_rows((NB, D))] * 5 +
             [_full((D, D)), _full((D, D)), _full((1, D)), _full((D, D)),
              _full((1, D)), _full((1, D)), _full((1, D)),
              _full((D, 1)), _full((1, 1))],
    out_specs=_rows((NB, 1)),
    out_shape=jax.ShapeDtypeStruct((N, 1), jnp.float32),
)


def kernel(x, pos, edge_index, edge_attr, emb_w, emb_b, edge_w1, edge_b1,
           edge_w2, edge_b2, att_w, att_b, node_w1, node_b1, node_w2,
           node_b2, ln_g, ln_b, out_w, out_b):
    src = edge_index[0]
    dst = edge_index[1]
    zeros = jnp.zeros((NPAD, D), jnp.float32)
    posp = jnp.pad(pos, ((0, 0), (0, D - 3)))

    relw = _relgather_call(posp, dst, src)
    h, P, R = _init_call(x, emb_w, emb_b.reshape(1, D),
                         edge_w1[0, :D], edge_w1[0, D:2 * D],
                         edge_b1[0].reshape(1, D))
    dists = [None, None]
    for l in range(L):
        eargs = (edge_w1[l, 2 * D].reshape(1, D), edge_w1[l, 2 * D + 1:],
                 edge_w2[l], edge_b2[l].reshape(1, D),
                 att_w[l], att_b[l].reshape(1, 1))
        parts = []
        for hf in range(2):
            G = _gather_calls[hf](P, R, dst, src)
            if l == 0:
                ma, dists[hf] = _edge_calls[hf][0](G, relw, edge_attr, *eargs)
            else:
                ma = _edge_calls[hf][1](G, dists[hf], edge_attr, *eargs)
            parts.append(_scatter_calls[hf](ma, dst, zeros))
        p00, p01 = parts[0][0], parts[0][1]
        p10, p11 = parts[1][0], parts[1][1]
        nargs = (node_w1[l, :D], node_w1[l, D:], node_b1[l].reshape(1, D),
                 node_w2[l], node_b2[l].reshape(1, D),
                 ln_g[l].reshape(1, D), ln_b[l].reshape(1, D))
        if l < L - 1:
            h, P, R = _node_mid_call(
                h, p00, p01, p10, p11, *nargs,
                edge_w1[l + 1, :D], edge_w1[l + 1, D:2 * D],
                edge_b1[l + 1].reshape(1, D))
        else:
            y = _node_last_call(h, p00, p01, p10, p11, *nargs,
                                out_w, out_b.reshape(1, 1))
    return y


# speedup vs baseline: 4.0700x; 1.0491x over previous
import functools

import jax
import jax.numpy as jnp
from jax import lax
from jax.experimental import pallas as pl
from jax.experimental.pallas import tpu as pltpu
from jax.experimental.pallas import tpu_sc as plsc

N = 10000
E = 320000
E2 = E // 2
D = 128
DE = 16
L = 4
NB = 1000
EB = 2000
NW = 32
NPAD = 10240
SUP = 200
CH = 128

_silu = jax.nn.silu


def _init_body(x_ref, ew_ref, eb_ref, w1a_ref, w1b_ref, b1_ref,
               h_ref, p_ref, r_ref):
    h = jnp.maximum(x_ref[...] @ ew_ref[...] + eb_ref[...], 0.0)
    h_ref[...] = h
    p_ref[...] = h @ w1a_ref[...] + b1_ref[...]
    r_ref[...] = h @ w1b_ref[...]


_full = lambda shape: pl.BlockSpec(shape, lambda i: (0,) * len(shape))
_rows = lambda shape: pl.BlockSpec(shape, lambda i: (i,) + (0,) * (len(shape) - 1))

_init_call = pl.pallas_call(
    _init_body,
    grid=(N // NB,),
    in_specs=[_rows((NB, D)), _full((D, D)), _full((1, D)),
              _full((D, D)), _full((D, D)), _full((1, D))],
    out_specs=[_rows((NB, D)), _rows((NB, D)), _rows((NB, D))],
    out_shape=[jax.ShapeDtypeStruct((N, D), jnp.float32)] * 3,
)


_sc_mesh = plsc.VectorSubcoreMesh(core_axis_name="c", subcore_axis_name="s")


@functools.partial(
    pl.kernel,
    mesh=_sc_mesh,
    out_type=jax.ShapeDtypeStruct((E, D), jnp.float32),
    scratch_types=[
        pltpu.VMEM((128,), jnp.int32),
        pltpu.VMEM((72,), jnp.int32),
        pltpu.VMEM((128,), jnp.int32),
        pltpu.VMEM((72,), jnp.int32),
        pltpu.VMEM((SUP, D), jnp.float32),
        pltpu.VMEM((SUP, D), jnp.float32),
        pltpu.SemaphoreType.DMA,
    ],
)
def _relgather_call(posp_hbm, dst_hbm, src_hbm, rel_hbm,
                    dia, dib, sia, sib, ba, bb, sem):
    wid = lax.axis_index("s") * 2 + lax.axis_index("c")
    base = wid * (E // NW)

    def body(j, carry):
        off = base + j * SUP
        pltpu.sync_copy(dst_hbm.at[pl.ds(off, 128)], dia)
        pltpu.sync_copy(dst_hbm.at[pl.ds(off + 128, 72)], dib)
        pltpu.sync_copy(src_hbm.at[pl.ds(off, 128)], sia)
        pltpu.sync_copy(src_hbm.at[pl.ds(off + 128, 72)], sib)
        cps = (pltpu.make_async_copy(posp_hbm.at[dia], ba.at[pl.ds(0, 128)], sem),
               pltpu.make_async_copy(posp_hbm.at[dib], ba.at[pl.ds(128, 72)], sem),
               pltpu.make_async_copy(posp_hbm.at[sia], bb.at[pl.ds(0, 128)], sem),
               pltpu.make_async_copy(posp_hbm.at[sib], bb.at[pl.ds(128, 72)], sem))
        for cp in cps:
            cp.start()
        for cp in cps:
            cp.wait()

        def sub_row(r, c2):
            sl = pl.ds(0, 16)
            ba[r, sl] = ba[r, sl] - bb[r, sl]
            return c2

        lax.fori_loop(0, SUP, sub_row, 0)
        pltpu.sync_copy(ba, rel_hbm.at[pl.ds(off, SUP)])
        return carry

    lax.fori_loop(0, E // NW // SUP, body, 0)


def _make_gather(b0):
    epw = E2 // NW
    ns = epw // SUP

    @functools.partial(
        pl.kernel,
        mesh=_sc_mesh,
        out_type=jax.ShapeDtypeStruct((E2, D), jnp.float32),
        scratch_types=(
            [pltpu.VMEM((128,), jnp.int32), pltpu.VMEM((72,), jnp.int32),
             pltpu.VMEM((128,), jnp.int32), pltpu.VMEM((72,), jnp.int32)] * 2
            + [pltpu.VMEM((SUP, D), jnp.float32)] * 4
            + [pltpu.SemaphoreType.DMA] * 6
        ),
    )
    def gather(p_hbm, r_hbm, dst_hbm, src_hbm, g_hbm,
               dia0, dib0, sia0, sib0, dia1, dib1, sia1, sib1,
               pb0, pb1, rb0, rb1, is0, is1, gs0, gs1, ws0, ws1):
        wid = lax.axis_index("s") * 2 + lax.axis_index("c")
        wbase = wid * epw
        idx = ((dia0, dib0, sia0, sib0), (dia1, dib1, sia1, sib1))
        pbs = (pb0, pb1)
        rbs = (rb0, rb1)
        isems = (is0, is1)
        gsems = (gs0, gs1)
        wsems = (ws0, ws1)

        def idx_copies(b, j):
            off = b0 + wbase + j * SUP
            a = idx[b]
            return (pltpu.make_async_copy(dst_hbm.at[pl.ds(off, 128)], a[0], isems[b]),
                    pltpu.make_async_copy(dst_hbm.at[pl.ds(off + 128, 72)], a[1], isems[b]),
                    pltpu.make_async_copy(src_hbm.at[pl.ds(off, 128)], a[2], isems[b]),
                    pltpu.make_async_copy(src_hbm.at[pl.ds(off + 128, 72)], a[3], isems[b]))

        def load_idx(b, j):
            for cp in idx_copies(b, j):
                cp.start()

        def wait_idx(b):
            for cp in idx_copies(b, 0):
                cp.wait()

        def gather_copies(b):
            a = idx[b]
            return (pltpu.make_async_copy(p_hbm.at[a[0]], pbs[b].at[pl.ds(0, 128)], gsems[b]),
                    pltpu.make_async_copy(p_hbm.at[a[1]], pbs[b].at[pl.ds(128, 72)], gsems[b]),
                    pltpu.make_async_copy(r_hbm.at[a[2]], rbs[b].at[pl.ds(0, 128)], gsems[b]),
                    pltpu.make_async_copy(r_hbm.at[a[3]], rbs[b].at[pl.ds(128, 72)], gsems[b]))

        def wb(b, j):
            return pltpu.make_async_copy(
                pbs[b], g_hbm.at[pl.ds(wbase + j * SUP, SUP)], wsems[b])

        def add_rows(b):
            def add_row(r, c2):
                for q in range(2):
                    for s in range(D // 16):
                        sl = pl.ds(s * 16, 16)
                        pbs[b][2 * r + q, sl] = (pbs[b][2 * r + q, sl]
                                                 + rbs[b][2 * r + q, sl])
                return c2

            lax.fori_loop(0, SUP // 2, add_row, 0)

        load_idx(0, 0)
        wait_idx(0)
        for cp in gather_copies(0):
            cp.start()
        load_idx(1, 1)

        def pair(t, carry):
            for b in (0, 1):
                j = 2 * t + b
                nb = 1 - b
                for cp in gather_copies(b):
                    cp.wait()
                add_rows(b)
                wb(b, j).start()

                @pl.when(j + 1 < ns)
                def _():
                    wait_idx(nb)

                    @pl.when(j >= 1)
                    def _():
                        wb(nb, 0).wait()

                    for cp in gather_copies(nb):
                        cp.start()

                @pl.when(j + 2 < ns)
                def _():
                    load_idx(b, j + 2)

            return carry

        lax.fori_loop(0, ns // 2, pair, 0)
        if ns % 2 == 1:
            for cp in gather_copies(0):
                cp.wait()
            add_rows(0)
            wb(0, ns - 1).start()
        wb(0, 0).wait()
        wb(1, 0).wait()

    return gather


_gather_calls = [_make_gather(0), _make_gather(E2)]


def _make_scatter(b0):
    nchunk = E2 // CH
    percore = nchunk // 2
    nmy = percore // 16

    @functools.partial(
        pl.kernel,
        mesh=_sc_mesh,
        out_type=jax.ShapeDtypeStruct((2, NPAD, D), jnp.float32),
        scratch_types=[
            pltpu.VMEM((CH,), jnp.int32), pltpu.VMEM((CH,), jnp.int32),
            pltpu.VMEM((CH, D), jnp.float32), pltpu.VMEM((CH, D), jnp.float32),
            pltpu.SemaphoreType.DMA, pltpu.SemaphoreType.DMA,
            pltpu.VMEM_SHARED((NPAD, D), jnp.float32),
        ],
    )
    def scatter(ma_hbm, dst_hbm, zero_hbm, out_hbm,
                di0, di1, rb0, rb1, ls0, ls1, acc_sh):
        cid = lax.axis_index("c")
        sid = lax.axis_index("s")
        dis = (di0, di1)
        rbs = (rb0, rb1)
        lsems = (ls0, ls1)
        mybase = cid * percore + sid * nmy

        def load_copies(b, j):
            loff = (mybase + j) * CH
            return (pltpu.make_async_copy(dst_hbm.at[pl.ds(b0 + loff, CH)],
                                          dis[b], lsems[b]),
                    pltpu.make_async_copy(ma_hbm.at[pl.ds(loff, CH)],
                                          rbs[b], lsems[b]))

        for cp in load_copies(0, 0):
            cp.start()
        rows = NPAD // 16
        sl_mine = pl.ds(sid * rows, rows)
        pltpu.sync_copy(zero_hbm.at[sl_mine], acc_sh.at[sl_mine])
        plsc.subcore_barrier()

        def step(j_is_last, b, j):
            for cp in load_copies(b, 0):
                cp.wait()
            if not j_is_last:
                for cp in load_copies(1 - b, j + 1):
                    cp.start()
            pltpu.sync_copy(rbs[b], acc_sh.at[dis[b]], add=True)

        def pair(t, carry):
            for b in (0, 1):
                step(False, b, 2 * t + b)
            return carry

        lax.fori_loop(0, nmy // 2, pair, 0)
        step(True, (nmy - 1) % 2, nmy - 1)

        @pl.when(sid == 0)
        def _():
            loff = (cid * percore + percore - 1) * CH
            pltpu.sync_copy(dst_hbm.at[pl.ds(b0 + loff, CH)], di0)
            pltpu.sync_copy(ma_hbm.at[pl.ds(loff, CH)], rb0)
            pltpu.sync_copy(rb0, acc_sh.at[di0], add=True)

        plsc.subcore_barrier()
        pltpu.sync_copy(acc_sh.at[sl_mine], out_hbm.at[cid, sl_mine])

    return scatter


_scatter_calls = [_make_scatter(0), _make_scatter(E2)]


def _edge_core(pre0, ea, w1c_ref, w1d_ref, w2_ref, b2_ref, aw_ref, ab_ref,
               dist):
    pre = pre0 + dist * w1c_ref[...] + ea @ w1d_ref[...]
    m = _silu(_silu(pre) @ w2_ref[...] + b2_ref[...])
    a = jax.nn.sigmoid(m @ aw_ref[...] + ab_ref[...])
    return m * a


def _make_edge_calls(h):
    off = h * (E2 // EB)
    _erows = lambda w: pl.BlockSpec((EB, w), lambda i: (off + i, 0))

    def edge0_body(g_ref, rel_ref, ea_ref, w1c_ref, w1d_ref, w2_ref, b2_ref,
                   aw_ref, ab_ref, ma_ref, dist_ref):
        rel = rel_ref[...]
        dist = jnp.sqrt(jnp.sum(rel * rel, axis=1, keepdims=True) + 1e-12)
        dist_ref[...] = dist
        ma_ref[...] = _edge_core(g_ref[...], ea_ref[...], w1c_ref, w1d_ref,
                                 w2_ref, b2_ref, aw_ref, ab_ref, dist)

    edge0 = pl.pallas_call(
        edge0_body,
        grid=(E2 // EB,),
        in_specs=[_rows((EB, D)), _erows(D), _erows(DE), _full((1, D)),
                  _full((DE, D)), _full((D, D)), _full((1, D)),
                  _full((D, 1)), _full((1, 1))],
        out_specs=[_rows((EB, D)), _rows((EB, 1))],
        out_shape=[jax.ShapeDtypeStruct((E2, D), jnp.float32),
                   jax.ShapeDtypeStruct((E2, 1), jnp.float32)],
    )

    def edge_body(g_ref, dist_ref, ea_ref, w1c_ref, w1d_ref, w2_ref, b2_ref,
                  aw_ref, ab_ref, ma_ref):
        ma_ref[...] = _edge_core(g_ref[...], ea_ref[...], w1c_ref, w1d_ref,
                                 w2_ref, b2_ref, aw_ref, ab_ref,
                                 dist_ref[...])

    edge = pl.pallas_call(
        edge_body,
        grid=(E2 // EB,),
        in_specs=[_rows((EB, D)), _rows((EB, 1)), _erows(DE), _full((1, D)),
                  _full((DE, D)), _full((D, D)), _full((1, D)),
                  _full((D, 1)), _full((1, 1))],
        out_specs=_rows((EB, D)),
        out_shape=jax.ShapeDtypeStruct((E2, D), jnp.float32),
    )

    return edge0, edge


_edge_calls = [_make_edge_calls(0), _make_edge_calls(1)]


def _node_common(h_ref, parts, nw1h_ref, nw1a_ref, nb1_ref,
                 nw2_ref, nb2_ref, g_ref, b_ref):
    h = h_ref[...]
    aggr = ((parts[0][...] + parts[1][...])
            + (parts[2][...] + parts[3][...]))
    nin1 = h @ nw1h_ref[...] + aggr @ nw1a_ref[...] + nb1_ref[...]
    u = h + _silu(nin1) @ nw2_ref[...] + nb2_ref[...]
    mu = jnp.mean(u, axis=1, keepdims=True)
    var = jnp.mean((u - mu) ** 2, axis=1, keepdims=True)
    return (u - mu) / jnp.sqrt(var + 1e-5) * g_ref[...] + b_ref[...]


def _node_mid_body(h_ref, p00, p01, p10, p11, nw1h_ref, nw1a_ref,
                   nb1_ref, nw2_ref, nb2_ref, g_ref, b_ref,
                   w1a_ref, w1b_ref, b1_ref, h_out, p_out, r_out):
    hn = _node_common(h_ref, (p00, p01, p10, p11), nw1h_ref, nw1a_ref,
                      nb1_ref, nw2_ref, nb2_ref, g_ref, b_ref)
    h_out[...] = hn
    p_out[...] = hn @ w1a_ref[...] + b1_ref[...]
    r_out[...] = hn @ w1b_ref[...]


_node_mid_call = pl.pallas_call(
    _node_mid_body,
    grid=(N // NB,),
    in_specs=[_rows((NB, D))] * 5 +
             [_full((D, D)), _full((D, D)), _full((1, D)), _full((D, D)),
              _full((1, D)), _full((1, D)), _full((1, D)),
              _full((D, D)), _full((D, D)), _full((1, D))],
    out_specs=[_rows((NB, D)), _rows((NB, D)), _rows((NB, D))],
    out_shape=[jax.ShapeDtypeStruct((N, D), jnp.float32)] * 3,
)


def _node_last_body(h_ref, p00, p01, p10, p11, nw1h_ref, nw1a_ref, nb1_ref,
                    nw2_ref, nb2_ref, g_ref, b_ref, ow_ref, ob_ref, y_out):
    hn = _node_common(h_ref, (p00, p01, p10, p11), nw1h_ref, nw1a_ref,
                      nb1_ref, nw2_ref, nb2_ref, g_ref, b_ref)
    y_out[...] = hn @ ow_ref[...] + ob_ref[...]


_node_last_call = pl.pallas_call(
    _node_last_body,
    grid=(N // NB,),
    in_specs=[_rows((NB, D))] * 5 +
             [_full((D, D)), _full((D, D)), _full((1, D)), _full((D, D)),
              _full((1, D)), _full((1, D)), _full((1, D)),
              _full((D, 1)), _full((1, 1))],
    out_specs=_rows((NB, 1)),
    out_shape=jax.ShapeDtypeStruct((N, 1), jnp.float32),
)


def kernel(x, pos, edge_index, edge_attr, emb_w, emb_b, edge_w1, edge_b1,
           edge_w2, edge_b2, att_w, att_b, node_w1, node_b1, node_w2,
           node_b2, ln_g, ln_b, out_w, out_b):
    src = edge_index[0]
    dst = edge_index[1]
    zeros = jnp.zeros((NPAD, D), jnp.float32)
    posp = jnp.pad(pos, ((0, 0), (0, D - 3)))

    relw = _relgather_call(posp, dst, src)
    h, P, R = _init_call(x, emb_w, emb_b.reshape(1, D),
                         edge_w1[0, :D], edge_w1[0, D:2 * D],
                         edge_b1[0].reshape(1, D))
    dists = [None, None]
    for l in range(L):
        eargs = (edge_w1[l, 2 * D].reshape(1, D), edge_w1[l, 2 * D + 1:],
                 edge_w2[l], edge_b2[l].reshape(1, D),
                 att_w[l], att_b[l].reshape(1, 1))
        parts = []
        for hf in range(2):
            G = _gather_calls[hf](P, R, dst, src)
            if l == 0:
                ma, dists[hf] = _edge_calls[hf][0](G, relw, edge_attr, *eargs)
            else:
                ma = _edge_calls[hf][1](G, dists[hf], edge_attr, *eargs)
            parts.append(_scatter_calls[hf](ma, dst, zeros))
        p00, p01 = parts[0][0], parts[0][1]
        p10, p11 = parts[1][0], parts[1][1]
        nargs = (node_w1[l, :D], node_w1[l, D:], node_b1[l].reshape(1, D),
                 node_w2[l], node_b2[l].reshape(1, D),
                 ln_g[l].reshape(1, D), ln_b[l].reshape(1, D))
        if l < L - 1:
            h, P, R = _node_mid_call(
                h, p00, p01, p10, p11, *nargs,
                edge_w1[l + 1, :D], edge_w1[l + 1, D:2 * D],
                edge_b1[l + 1].reshape(1, D))
        else:
            y = _node_last_call(h, p00, p01, p10, p11, *nargs,
                                out_w, out_b.reshape(1, 1))
    return y


# speedup vs baseline: 4.1382x; 1.0168x over previous
import functools

import jax
import jax.numpy as jnp
from jax import lax
from jax.experimental import pallas as pl
from jax.experimental.pallas import tpu as pltpu
from jax.experimental.pallas import tpu_sc as plsc

N = 10000
E = 320000
E2 = E // 2
D = 128
DE = 16
L = 4
NB = 1000
EB = 2000
NW = 32
NPAD = 10240
SUP = 200
CH = 128

_silu = jax.nn.silu


def _init_body(x_ref, ew_ref, eb_ref, w1a_ref, w1b_ref, b1_ref,
               h_ref, p_ref, r_ref):
    h = jnp.maximum(x_ref[...] @ ew_ref[...] + eb_ref[...], 0.0)
    h_ref[...] = h
    p_ref[...] = h @ w1a_ref[...] + b1_ref[...]
    r_ref[...] = h @ w1b_ref[...]


_full = lambda shape: pl.BlockSpec(shape, lambda i: (0,) * len(shape))
_rows = lambda shape: pl.BlockSpec(shape, lambda i: (i,) + (0,) * (len(shape) - 1))

_init_call = pl.pallas_call(
    _init_body,
    grid=(N // NB,),
    in_specs=[_rows((NB, D)), _full((D, D)), _full((1, D)),
              _full((D, D)), _full((D, D)), _full((1, D))],
    out_specs=[_rows((NB, D)), _rows((NB, D)), _rows((NB, D))],
    out_shape=[jax.ShapeDtypeStruct((N, D), jnp.float32)] * 3,
)


_sc_mesh = plsc.VectorSubcoreMesh(core_axis_name="c", subcore_axis_name="s")


def _make_gather(b0, etot=E2, subtract=False):
    epw = etot // NW
    ns = epw // SUP

    @functools.partial(
        pl.kernel,
        mesh=_sc_mesh,
        out_type=jax.ShapeDtypeStruct((etot, D), jnp.float32),
        scratch_types=(
            [pltpu.VMEM((128,), jnp.int32), pltpu.VMEM((72,), jnp.int32),
             pltpu.VMEM((128,), jnp.int32), pltpu.VMEM((72,), jnp.int32)] * 2
            + [pltpu.VMEM((SUP, D), jnp.float32)] * 4
            + [pltpu.SemaphoreType.DMA] * 6
        ),
    )
    def gather(p_hbm, r_hbm, dst_hbm, src_hbm, g_hbm,
               dia0, dib0, sia0, sib0, dia1, dib1, sia1, sib1,
               pb0, pb1, rb0, rb1, is0, is1, gs0, gs1, ws0, ws1):
        wid = lax.axis_index("s") * 2 + lax.axis_index("c")
        wbase = wid * epw
        idx = ((dia0, dib0, sia0, sib0), (dia1, dib1, sia1, sib1))
        pbs = (pb0, pb1)
        rbs = (rb0, rb1)
        isems = (is0, is1)
        gsems = (gs0, gs1)
        wsems = (ws0, ws1)

        def idx_copies(b, j):
            off = b0 + wbase + j * SUP
            a = idx[b]
            return (pltpu.make_async_copy(dst_hbm.at[pl.ds(off, 128)], a[0], isems[b]),
                    pltpu.make_async_copy(dst_hbm.at[pl.ds(off + 128, 72)], a[1], isems[b]),
                    pltpu.make_async_copy(src_hbm.at[pl.ds(off, 128)], a[2], isems[b]),
                    pltpu.make_async_copy(src_hbm.at[pl.ds(off + 128, 72)], a[3], isems[b]))

        def load_idx(b, j):
            for cp in idx_copies(b, j):
                cp.start()

        def wait_idx(b):
            for cp in idx_copies(b, 0):
                cp.wait()

        def gather_copies(b):
            a = idx[b]
            return (pltpu.make_async_copy(p_hbm.at[a[0]], pbs[b].at[pl.ds(0, 128)], gsems[b]),
                    pltpu.make_async_copy(p_hbm.at[a[1]], pbs[b].at[pl.ds(128, 72)], gsems[b]),
                    pltpu.make_async_copy(r_hbm.at[a[2]], rbs[b].at[pl.ds(0, 128)], gsems[b]),
                    pltpu.make_async_copy(r_hbm.at[a[3]], rbs[b].at[pl.ds(128, 72)], gsems[b]))

        def wb(b, j):
            return pltpu.make_async_copy(
                pbs[b], g_hbm.at[pl.ds(wbase + j * SUP, SUP)], wsems[b])

        def add_rows(b):
            def add_row(r, c2):
                for q in range(2):
                    for s in range(D // 16):
                        sl = pl.ds(s * 16, 16)
                        if subtract:
                            pbs[b][2 * r + q, sl] = (pbs[b][2 * r + q, sl]
                                                     - rbs[b][2 * r + q, sl])
                        else:
                            pbs[b][2 * r + q, sl] = (pbs[b][2 * r + q, sl]
                                                     + rbs[b][2 * r + q, sl])
                return c2

            lax.fori_loop(0, SUP // 2, add_row, 0)

        load_idx(0, 0)
        wait_idx(0)
        for cp in gather_copies(0):
            cp.start()
        load_idx(1, 1)

        def pair(t, carry):
            for b in (0, 1):
                j = 2 * t + b
                nb = 1 - b
                for cp in gather_copies(b):
                    cp.wait()
                add_rows(b)
                wb(b, j).start()

                @pl.when(j + 1 < ns)
                def _():
                    wait_idx(nb)

                    @pl.when(j >= 1)
                    def _():
                        wb(nb, 0).wait()

                    for cp in gather_copies(nb):
                        cp.start()

                @pl.when(j + 2 < ns)
                def _():
                    load_idx(b, j + 2)

            return carry

        lax.fori_loop(0, ns // 2, pair, 0)
        if ns % 2 == 1:
            for cp in gather_copies(0):
                cp.wait()
            add_rows(0)
            wb(0, ns - 1).start()
        wb(0, 0).wait()
        wb(1, 0).wait()

    return gather


_gather_calls = [_make_gather(0), _make_gather(E2)]
_relgather_call = _make_gather(0, E, True)


def _make_scatter(b0):
    nchunk = E2 // CH
    percore = nchunk // 2
    nmy = percore // 16

    @functools.partial(
        pl.kernel,
        mesh=_sc_mesh,
        out_type=jax.ShapeDtypeStruct((2, NPAD, D), jnp.float32),
        scratch_types=[
            pltpu.VMEM((CH,), jnp.int32), pltpu.VMEM((CH,), jnp.int32),
            pltpu.VMEM((CH, D), jnp.float32), pltpu.VMEM((CH, D), jnp.float32),
            pltpu.SemaphoreType.DMA, pltpu.SemaphoreType.DMA,
            pltpu.VMEM_SHARED((NPAD, D), jnp.float32),
        ],
    )
    def scatter(ma_hbm, dst_hbm, zero_hbm, out_hbm,
                di0, di1, rb0, rb1, ls0, ls1, acc_sh):
        cid = lax.axis_index("c")
        sid = lax.axis_index("s")
        dis = (di0, di1)
        rbs = (rb0, rb1)
        lsems = (ls0, ls1)
        mybase = cid * percore + sid * nmy

        def load_copies(b, j):
            loff = (mybase + j) * CH
            return (pltpu.make_async_copy(dst_hbm.at[pl.ds(b0 + loff, CH)],
                                          dis[b], lsems[b]),
                    pltpu.make_async_copy(ma_hbm.at[pl.ds(loff, CH)],
                                          rbs[b], lsems[b]))

        for cp in load_copies(0, 0):
            cp.start()
        rows = NPAD // 16
        sl_mine = pl.ds(sid * rows, rows)
        pltpu.sync_copy(zero_hbm.at[sl_mine], acc_sh.at[sl_mine])
        plsc.subcore_barrier()

        def step(j_is_last, b, j):
            for cp in load_copies(b, 0):
                cp.wait()
            if not j_is_last:
                for cp in load_copies(1 - b, j + 1):
                    cp.start()
            pltpu.sync_copy(rbs[b], acc_sh.at[dis[b]], add=True)

        def pair(t, carry):
            for b in (0, 1):
                step(False, b, 2 * t + b)
            return carry

        lax.fori_loop(0, nmy // 2, pair, 0)
        step(True, (nmy - 1) % 2, nmy - 1)

        @pl.when(sid == 0)
        def _():
            loff = (cid * percore + percore - 1) * CH
            pltpu.sync_copy(dst_hbm.at[pl.ds(b0 + loff, CH)], di0)
            pltpu.sync_copy(ma_hbm.at[pl.ds(loff, CH)], rb0)
            pltpu.sync_copy(rb0, acc_sh.at[di0], add=True)

        plsc.subcore_barrier()
        pltpu.sync_copy(acc_sh.at[sl_mine], out_hbm.at[cid, sl_mine])

    return scatter


_scatter_calls = [_make_scatter(0), _make_scatter(E2)]


def _edge_core(pre0, ea, w1c_ref, w1d_ref, w2_ref, b2_ref, aw_ref, ab_ref,
               dist):
    pre = pre0 + dist * w1c_ref[...] + ea @ w1d_ref[...]
    h1 = _silu(pre).astype(jnp.bfloat16)
    w2 = w2_ref[...].astype(jnp.bfloat16)
    m = _silu(jnp.dot(h1, w2, preferred_element_type=jnp.float32)
              + b2_ref[...])
    a = jax.nn.sigmoid(m @ aw_ref[...] + ab_ref[...])
    return m * a


def _make_edge_calls(h):
    off = h * (E2 // EB)
    _erows = lambda w: pl.BlockSpec((EB, w), lambda i: (off + i, 0))

    def edge0_body(g_ref, rel_ref, ea_ref, w1c_ref, w1d_ref, w2_ref, b2_ref,
                   aw_ref, ab_ref, ma_ref, dist_ref):
        rel = rel_ref[...]
        dist = jnp.sqrt(jnp.sum(rel * rel, axis=1, keepdims=True) + 1e-12)
        dist_ref[...] = dist
        ma_ref[...] = _edge_core(g_ref[...], ea_ref[...], w1c_ref, w1d_ref,
                                 w2_ref, b2_ref, aw_ref, ab_ref, dist)

    edge0 = pl.pallas_call(
        edge0_body,
        grid=(E2 // EB,),
        in_specs=[_rows((EB, D)), _erows(D), _erows(DE), _full((1, D)),
                  _full((DE, D)), _full((D, D)), _full((1, D)),
                  _full((D, 1)), _full((1, 1))],
        out_specs=[_rows((EB, D)), _rows((EB, 1))],
        out_shape=[jax.ShapeDtypeStruct((E2, D), jnp.float32),
                   jax.ShapeDtypeStruct((E2, 1), jnp.float32)],
    )

    def edge_body(g_ref, dist_ref, ea_ref, w1c_ref, w1d_ref, w2_ref, b2_ref,
                  aw_ref, ab_ref, ma_ref):
        ma_ref[...] = _edge_core(g_ref[...], ea_ref[...], w1c_ref, w1d_ref,
                                 w2_ref, b2_ref, aw_ref, ab_ref,
                                 dist_ref[...])

    edge = pl.pallas_call(
        edge_body,
        grid=(E2 // EB,),
        in_specs=[_rows((EB, D)), _rows((EB, 1)), _erows(DE), _full((1, D)),
                  _full((DE, D)), _full((D, D)), _full((1, D)),
                  _full((D, 1)), _full((1, 1))],
        out_specs=_rows((EB, D)),
        out_shape=jax.ShapeDtypeStruct((E2, D), jnp.float32),
    )

    return edge0, edge


_edge_calls = [_make_edge_calls(0), _make_edge_calls(1)]


def _node_common(h_ref, parts, nw1h_ref, nw1a_ref, nb1_ref,
                 nw2_ref, nb2_ref, g_ref, b_ref):
    h = h_ref[...]
    aggr = ((parts[0][...] + parts[1][...])
            + (parts[2][...] + parts[3][...]))
    nin1 = h @ nw1h_ref[...] + aggr @ nw1a_ref[...] + nb1_ref[...]
    u = h + _silu(nin1) @ nw2_ref[...] + nb2_ref[...]
    mu = jnp.mean(u, axis=1, keepdims=True)
    var = jnp.mean((u - mu) ** 2, axis=1, keepdims=True)
    return (u - mu) / jnp.sqrt(var + 1e-5) * g_ref[...] + b_ref[...]


def _node_mid_body(h_ref, p00, p01, p10, p11, nw1h_ref, nw1a_ref,
                   nb1_ref, nw2_ref, nb2_ref, g_ref, b_ref,
                   w1a_ref, w1b_ref, b1_ref, h_out, p_out, r_out):
    hn = _node_common(h_ref, (p00, p01, p10, p11), nw1h_ref, nw1a_ref,
                      nb1_ref, nw2_ref, nb2_ref, g_ref, b_ref)
    h_out[...] = hn
    p_out[...] = hn @ w1a_ref[...] + b1_ref[...]
    r_out[...] = hn @ w1b_ref[...]


_node_mid_call = pl.pallas_call(
    _node_mid_body,
    grid=(N // NB,),
    in_specs=[_rows((NB, D))] * 5 +
             [_full((D, D)), _full((D, D)), _full((1, D)), _full((D, D)),
              _full((1, D)), _full((1, D)), _full((1, D)),
              _full((D, D)), _full((D, D)), _full((1, D))],
    out_specs=[_rows((NB, D)), _rows((NB, D)), _rows((NB, D))],
    out_shape=[jax.ShapeDtypeStruct((N, D), jnp.float32)] * 3,
)


def _node_last_body(h_ref, p00, p01, p10, p11, nw1h_ref, nw1a_ref, nb1_ref,
                    nw2_ref, nb2_ref, g_ref, b_ref, ow_ref, ob_ref, y_out):
    hn = _node_common(h_ref, (p00, p01, p10, p11), nw1h_ref, nw1a_ref,
                      nb1_ref, nw2_ref, nb2_ref, g_ref, b_ref)
    y_out[...] = hn @ ow_ref[...] + ob_ref[...]


_node_last_call = pl.pallas_call(
    _node_last_body,
    grid=(N // NB,),
    in_specs=[_rows((NB, D))] * 5 +
             [_full((D, D)), _full((D, D)), _full((1, D)), _full((D, D)),
              _full((1, D)), _full((1, D)), _full((1, D)),
              _full((D, 1)), _full((1, 1))],
    out_specs=_rows((NB, 1)),
    out_shape=jax.ShapeDtypeStruct((N, 1), jnp.float32),
)


def kernel(x, pos, edge_index, edge_attr, emb_w, emb_b, edge_w1, edge_b1,
           edge_w2, edge_b2, att_w, att_b, node_w1, node_b1, node_w2,
           node_b2, ln_g, ln_b, out_w, out_b):
    src = edge_index[0]
    dst = edge_index[1]
    zeros = jnp.zeros((NPAD, D), jnp.float32)
    posp = jnp.pad(pos, ((0, 0), (0, D - 3)))

    relw = _relgather_call(posp, posp, dst, src)
    h, P, R = _init_call(x, emb_w, emb_b.reshape(1, D),
                         edge_w1[0, :D], edge_w1[0, D:2 * D],
                         edge_b1[0].reshape(1, D))
    dists = [None, None]
    for l in range(L):
        eargs = (edge_w1[l, 2 * D].reshape(1, D), edge_w1[l, 2 * D + 1:],
                 edge_w2[l], edge_b2[l].reshape(1, D),
                 att_w[l], att_b[l].reshape(1, 1))
        parts = []
        for hf in range(2):
            G = _gather_calls[hf](P, R, dst, src)
            if l == 0:
                ma, dists[hf] = _edge_calls[hf][0](G, relw, edge_attr, *eargs)
            else:
                ma = _edge_calls[hf][1](G, dists[hf], edge_attr, *eargs)
            parts.append(_scatter_calls[hf](ma, dst, zeros))
        p00, p01 = parts[0][0], parts[0][1]
        p10, p11 = parts[1][0], parts[1][1]
        nargs = (node_w1[l, :D], node_w1[l, D:], node_b1[l].reshape(1, D),
                 node_w2[l], node_b2[l].reshape(1, D),
                 ln_g[l].reshape(1, D), ln_b[l].reshape(1, D))
        if l < L - 1:
            h, P, R = _node_mid_call(
                h, p00, p01, p10, p11, *nargs,
                edge_w1[l + 1, :D], edge_w1[l + 1, D:2 * D],
                edge_b1[l + 1].reshape(1, D))
        else:
            y = _node_last_call(h, p00, p01, p10, p11, *nargs,
                                out_w, out_b.reshape(1, 1))
    return y


# speedup vs baseline: 4.4232x; 1.0689x over previous
import functools

import jax
import jax.numpy as jnp
from jax import lax
from jax.experimental import pallas as pl
from jax.experimental.pallas import tpu as pltpu
from jax.experimental.pallas import tpu_sc as plsc

N = 10000
E = 320000
E2 = E // 2
D = 128
DE = 16
L = 4
NB = 1000
EB = 4000
NW = 32
NPAD = 10240
SUP = 200
CH = 128

_silu = jax.nn.silu


def _init_body(x_ref, ew_ref, eb_ref, w1a_ref, w1b_ref, b1_ref,
               h_ref, p_ref, r_ref):
    h = jnp.maximum(x_ref[...] @ ew_ref[...] + eb_ref[...], 0.0)
    h_ref[...] = h
    p_ref[...] = h @ w1a_ref[...] + b1_ref[...]
    r_ref[...] = h @ w1b_ref[...]


_full = lambda shape: pl.BlockSpec(shape, lambda i: (0,) * len(shape))
_rows = lambda shape: pl.BlockSpec(shape, lambda i: (i,) + (0,) * (len(shape) - 1))

_init_call = pl.pallas_call(
    _init_body,
    grid=(N // NB,),
    in_specs=[_rows((NB, D)), _full((D, D)), _full((1, D)),
              _full((D, D)), _full((D, D)), _full((1, D))],
    out_specs=[_rows((NB, D)), _rows((NB, D)), _rows((NB, D))],
    out_shape=[jax.ShapeDtypeStruct((N, D), jnp.float32)] * 3,
)


_sc_mesh = plsc.VectorSubcoreMesh(core_axis_name="c", subcore_axis_name="s")


def _make_gather(b0, etot=E2, subtract=False):
    epw = etot // NW
    ns = epw // SUP

    @functools.partial(
        pl.kernel,
        mesh=_sc_mesh,
        out_type=jax.ShapeDtypeStruct((etot, D), jnp.float32),
        scratch_types=(
            [pltpu.VMEM((128,), jnp.int32), pltpu.VMEM((72,), jnp.int32),
             pltpu.VMEM((128,), jnp.int32), pltpu.VMEM((72,), jnp.int32)] * 2
            + [pltpu.VMEM((SUP, D), jnp.float32)] * 4
            + [pltpu.SemaphoreType.DMA] * 6
        ),
    )
    def gather(p_hbm, r_hbm, dst_hbm, src_hbm, g_hbm,
               dia0, dib0, sia0, sib0, dia1, dib1, sia1, sib1,
               pb0, pb1, rb0, rb1, is0, is1, gs0, gs1, ws0, ws1):
        wid = lax.axis_index("s") * 2 + lax.axis_index("c")
        wbase = wid * epw
        idx = ((dia0, dib0, sia0, sib0), (dia1, dib1, sia1, sib1))
        pbs = (pb0, pb1)
        rbs = (rb0, rb1)
        isems = (is0, is1)
        gsems = (gs0, gs1)
        wsems = (ws0, ws1)

        def idx_copies(b, j):
            off = b0 + wbase + j * SUP
            a = idx[b]
            return (pltpu.make_async_copy(dst_hbm.at[pl.ds(off, 128)], a[0], isems[b]),
                    pltpu.make_async_copy(dst_hbm.at[pl.ds(off + 128, 72)], a[1], isems[b]),
                    pltpu.make_async_copy(src_hbm.at[pl.ds(off, 128)], a[2], isems[b]),
                    pltpu.make_async_copy(src_hbm.at[pl.ds(off + 128, 72)], a[3], isems[b]))

        def load_idx(b, j):
            for cp in idx_copies(b, j):
                cp.start()

        def wait_idx(b):
            for cp in idx_copies(b, 0):
                cp.wait()

        def gather_copies(b):
            a = idx[b]
            return (pltpu.make_async_copy(p_hbm.at[a[0]], pbs[b].at[pl.ds(0, 128)], gsems[b]),
                    pltpu.make_async_copy(p_hbm.at[a[1]], pbs[b].at[pl.ds(128, 72)], gsems[b]),
                    pltpu.make_async_copy(r_hbm.at[a[2]], rbs[b].at[pl.ds(0, 128)], gsems[b]),
                    pltpu.make_async_copy(r_hbm.at[a[3]], rbs[b].at[pl.ds(128, 72)], gsems[b]))

        def wb(b, j):
            return pltpu.make_async_copy(
                pbs[b], g_hbm.at[pl.ds(wbase + j * SUP, SUP)], wsems[b])

        def add_rows(b):
            def add_row(r, c2):
                for q in range(2):
                    for s in range(D // 16):
                        sl = pl.ds(s * 16, 16)
                        if subtract:
                            pbs[b][2 * r + q, sl] = (pbs[b][2 * r + q, sl]
                                                     - rbs[b][2 * r + q, sl])
                        else:
                            pbs[b][2 * r + q, sl] = (pbs[b][2 * r + q, sl]
                                                     + rbs[b][2 * r + q, sl])
                return c2

            lax.fori_loop(0, SUP // 2, add_row, 0)

        load_idx(0, 0)
        wait_idx(0)
        for cp in gather_copies(0):
            cp.start()
        load_idx(1, 1)

        def pair(t, carry):
            for b in (0, 1):
                j = 2 * t + b
                nb = 1 - b
                for cp in gather_copies(b):
                    cp.wait()
                add_rows(b)
                wb(b, j).start()

                @pl.when(j + 1 < ns)
                def _():
                    wait_idx(nb)

                    @pl.when(j >= 1)
                    def _():
                        wb(nb, 0).wait()

                    for cp in gather_copies(nb):
                        cp.start()

                @pl.when(j + 2 < ns)
                def _():
                    load_idx(b, j + 2)

            return carry

        lax.fori_loop(0, ns // 2, pair, 0)
        if ns % 2 == 1:
            for cp in gather_copies(0):
                cp.wait()
            add_rows(0)
            wb(0, ns - 1).start()
        wb(0, 0).wait()
        wb(1, 0).wait()

    return gather


_gather_calls = [_make_gather(0), _make_gather(E2)]
_relgather_call = _make_gather(0, E, True)


def _make_scatter(b0):
    nchunk = E2 // CH
    percore = nchunk // 2
    nmy = percore // 16

    @functools.partial(
        pl.kernel,
        mesh=_sc_mesh,
        out_type=jax.ShapeDtypeStruct((2, NPAD, D), jnp.float32),
        scratch_types=[
            pltpu.VMEM((CH,), jnp.int32), pltpu.VMEM((CH,), jnp.int32),
            pltpu.VMEM((CH, D), jnp.float32), pltpu.VMEM((CH, D), jnp.float32),
            pltpu.SemaphoreType.DMA, pltpu.SemaphoreType.DMA,
            pltpu.VMEM_SHARED((NPAD, D), jnp.float32),
        ],
    )
    def scatter(ma_hbm, dst_hbm, zero_hbm, out_hbm,
                di0, di1, rb0, rb1, ls0, ls1, acc_sh):
        cid = lax.axis_index("c")
        sid = lax.axis_index("s")
        dis = (di0, di1)
        rbs = (rb0, rb1)
        lsems = (ls0, ls1)
        mybase = cid * percore + sid * nmy

        def load_copies(b, j):
            loff = (mybase + j) * CH
            return (pltpu.make_async_copy(dst_hbm.at[pl.ds(b0 + loff, CH)],
                                          dis[b], lsems[b]),
                    pltpu.make_async_copy(ma_hbm.at[pl.ds(loff, CH)],
                                          rbs[b], lsems[b]))

        for cp in load_copies(0, 0):
            cp.start()
        rows = NPAD // 16
        sl_mine = pl.ds(sid * rows, rows)
        pltpu.sync_copy(zero_hbm.at[sl_mine], acc_sh.at[sl_mine])
        plsc.subcore_barrier()

        def step(j_is_last, b, j):
            for cp in load_copies(b, 0):
                cp.wait()
            if not j_is_last:
                for cp in load_copies(1 - b, j + 1):
                    cp.start()
            pltpu.sync_copy(rbs[b], acc_sh.at[dis[b]], add=True)

        def pair(t, carry):
            for b in (0, 1):
                step(False, b, 2 * t + b)
            return carry

        lax.fori_loop(0, nmy // 2, pair, 0)
        step(True, (nmy - 1) % 2, nmy - 1)

        @pl.when(sid == 0)
        def _():
            loff = (cid * percore + percore - 1) * CH
            pltpu.sync_copy(dst_hbm.at[pl.ds(b0 + loff, CH)], di0)
            pltpu.sync_copy(ma_hbm.at[pl.ds(loff, CH)], rb0)
            pltpu.sync_copy(rb0, acc_sh.at[di0], add=True)

        plsc.subcore_barrier()
        pltpu.sync_copy(acc_sh.at[sl_mine], out_hbm.at[cid, sl_mine])

    return scatter


_scatter_calls = [_make_scatter(0), _make_scatter(E2)]


def _edge_core(pre0, ea, w1c_ref, w1d_ref, w2_ref, b2_ref, aw_ref, ab_ref,
               dist):
    pre = pre0 + dist * w1c_ref[...] + ea @ w1d_ref[...]
    h1 = _silu(pre.astype(jnp.bfloat16))
    w2 = w2_ref[...].astype(jnp.bfloat16)
    m = _silu(jnp.dot(h1, w2, preferred_element_type=jnp.float32)
              + b2_ref[...])
    a = jax.nn.sigmoid(m @ aw_ref[...] + ab_ref[...])
    return m * a


def _make_edge_calls(h):
    off = h * (E2 // EB)
    _erows = lambda w: pl.BlockSpec((EB, w), lambda i: (off + i, 0))

    def edge0_body(g_ref, rel_ref, ea_ref, w1c_ref, w1d_ref, w2_ref, b2_ref,
                   aw_ref, ab_ref, ma_ref, dist_ref):
        rel = rel_ref[...]
        dist = jnp.sqrt(jnp.sum(rel * rel, axis=1, keepdims=True) + 1e-12)
        dist_ref[...] = dist
        ma_ref[...] = _edge_core(g_ref[...], ea_ref[...], w1c_ref, w1d_ref,
                                 w2_ref, b2_ref, aw_ref, ab_ref, dist)

    edge0 = pl.pallas_call(
        edge0_body,
        grid=(E2 // EB,),
        in_specs=[_rows((EB, D)), _erows(D), _erows(DE), _full((1, D)),
                  _full((DE, D)), _full((D, D)), _full((1, D)),
                  _full((D, 1)), _full((1, 1))],
        out_specs=[_rows((EB, D)), _rows((EB, 1))],
        out_shape=[jax.ShapeDtypeStruct((E2, D), jnp.float32),
                   jax.ShapeDtypeStruct((E2, 1), jnp.float32)],
    )

    def edge_body(g_ref, dist_ref, ea_ref, w1c_ref, w1d_ref, w2_ref, b2_ref,
                  aw_ref, ab_ref, ma_ref):
        ma_ref[...] = _edge_core(g_ref[...], ea_ref[...], w1c_ref, w1d_ref,
                                 w2_ref, b2_ref, aw_ref, ab_ref,
                                 dist_ref[...])

    edge = pl.pallas_call(
        edge_body,
        grid=(E2 // EB,),
        in_specs=[_rows((EB, D)), _rows((EB, 1)), _erows(DE), _full((1, D)),
                  _full((DE, D)), _full((D, D)), _full((1, D)),
                  _full((D, 1)), _full((1, 1))],
        out_specs=_rows((EB, D)),
        out_shape=jax.ShapeDtypeStruct((E2, D), jnp.float32),
    )

    return edge0, edge


_edge_calls = [_make_edge_calls(0), _make_edge_calls(1)]


def _node_common(h_ref, parts, nw1h_ref, nw1a_ref, nb1_ref,
                 nw2_ref, nb2_ref, g_ref, b_ref):
    h = h_ref[...]
    aggr = ((parts[0][...] + parts[1][...])
            + (parts[2][...] + parts[3][...]))
    nin1 = h @ nw1h_ref[...] + aggr @ nw1a_ref[...] + nb1_ref[...]
    u = h + _silu(nin1) @ nw2_ref[...] + nb2_ref[...]
    mu = jnp.mean(u, axis=1, keepdims=True)
    var = jnp.mean((u - mu) ** 2, axis=1, keepdims=True)
    return (u - mu) / jnp.sqrt(var + 1e-5) * g_ref[...] + b_ref[...]


def _node_mid_body(h_ref, p00, p01, p10, p11, nw1h_ref, nw1a_ref,
                   nb1_ref, nw2_ref, nb2_ref, g_ref, b_ref,
                   w1a_ref, w1b_ref, b1_ref, h_out, p_out, r_out):
    hn = _node_common(h_ref, (p00, p01, p10, p11), nw1h_ref, nw1a_ref,
                      nb1_ref, nw2_ref, nb2_ref, g_ref, b_ref)
    h_out[...] = hn
    p_out[...] = hn @ w1a_ref[...] + b1_ref[...]
    r_out[...] = hn @ w1b_ref[...]


_node_mid_call = pl.pallas_call(
    _node_mid_body,
    grid=(N // NB,),
    in_specs=[_rows((NB, D))] * 5 +
             [_full((D, D)), _full((D, D)), _full((1, D)), _full((D, D)),
              _full((1, D)), _full((1, D)), _full((1, D)),
              _full((D, D)), _full((D, D)), _full((1, D))],
    out_specs=[_rows((NB, D)), _rows((NB, D)), _rows((NB, D))],
    out_shape=[jax.ShapeDtypeStruct((N, D), jnp.float32)] * 3,
)


def _node_last_body(h_ref, p00, p01, p10, p11, nw1h_ref, nw1a_ref, nb1_ref,
                    nw2_ref, nb2_ref, g_ref, b_ref, ow_ref, ob_ref, y_out):
    hn = _node_common(h_ref, (p00, p01, p10, p11), nw1h_ref, nw1a_ref,
                      nb1_ref, nw2_ref, nb2_ref, g_ref, b_ref)
    y_out[...] = hn @ ow_ref[...] + ob_ref[...]


_node_last_call = pl.pallas_call(
    _node_last_body,
    grid=(N // NB,),
    in_specs=[_rows((NB, D))] * 5 +
             [_full((D, D)), _full((D, D)), _full((1, D)), _full((D, D)),
              _full((1, D)), _full((1, D)), _full((1, D)),
              _full((D, 1)), _full((1, 1))],
    out_specs=_rows((NB, 1)),
    out_shape=jax.ShapeDtypeStruct((N, 1), jnp.float32),
)


def kernel(x, pos, edge_index, edge_attr, emb_w, emb_b, edge_w1, edge_b1,
           edge_w2, edge_b2, att_w, att_b, node_w1, node_b1, node_w2,
           node_b2, ln_g, ln_b, out_w, out_b):
    src = edge_index[0]
    dst = edge_index[1]
    zeros = jnp.zeros((NPAD, D), jnp.float32)
    posp = jnp.pad(pos, ((0, 0), (0, D - 3)))

    relw = _relgather_call(posp, posp, dst, src)
    h, P, R = _init_call(x, emb_w, emb_b.reshape(1, D),
                         edge_w1[0, :D], edge_w1[0, D:2 * D],
                         edge_b1[0].reshape(1, D))
    dists = [None, None]
    for l in range(L):
        eargs = (edge_w1[l, 2 * D].reshape(1, D), edge_w1[l, 2 * D + 1:],
                 edge_w2[l], edge_b2[l].reshape(1, D),
                 att_w[l], att_b[l].reshape(1, 1))
        parts = []
        for hf in range(2):
            G = _gather_calls[hf](P, R, dst, src)
            if l == 0:
                ma, dists[hf] = _edge_calls[hf][0](G, relw, edge_attr, *eargs)
            else:
                ma = _edge_calls[hf][1](G, dists[hf], edge_attr, *eargs)
            parts.append(_scatter_calls[hf](ma, dst, zeros))
        p00, p01 = parts[0][0], parts[0][1]
        p10, p11 = parts[1][0], parts[1][1]
        nargs = (node_w1[l, :D], node_w1[l, D:], node_b1[l].reshape(1, D),
                 node_w2[l], node_b2[l].reshape(1, D),
                 ln_g[l].reshape(1, D), ln_b[l].reshape(1, D))
        if l < L - 1:
            h, P, R = _node_mid_call(
                h, p00, p01, p10, p11, *nargs,
                edge_w1[l + 1, :D], edge_w1[l + 1, D:2 * D],
                edge_b1[l + 1].reshape(1, D))
        else:
            y = _node_last_call(h, p00, p01, p10, p11, *nargs,
                                out_w, out_b.reshape(1, 1))
    return y
